# baseline (speedup 1.0000x reference)
import functools

import jax
import jax.numpy as jnp
from jax import lax
from jax.experimental import pallas as pl
from jax.experimental.pallas import tpu as pltpu, tpu_sc as plsc

_NC = 2
_NS = 16
_NW = _NC * _NS
_L = 16
_CH = 128


def _sc_body(x_hbm, wb_hbm, out_hbm, xbuf, obuf, wbv):
    rows = x_hbm.shape[0]
    rows_per_w = rows // _NW
    wid = lax.axis_index("s") * _NC + lax.axis_index("c")
    base = wid * rows_per_w

    pltpu.sync_copy(wb_hbm, wbv)

    iota = lax.iota(jnp.int32, _L)

    def do_chunk(c, _):
        pltpu.sync_copy(x_hbm.at[pl.ds(base + c * _CH, _CH)], xbuf)
        for g in range(_CH // _L):
            row_ids = iota + (g * _L)

            def fstep(off):
                def step(f, accs):
                    col = jnp.full((_L,), 0, jnp.int32) + (f + off)
                    xv = plsc.load_gather(xbuf, [row_ids, col])
                    return tuple(
                        accs[d] + xv * wbv[(f + off) * _L + d]
                        for d in range(_L)
                    )
                return step

            zero = jnp.zeros((_L,), jnp.float32)
            acc_r = lax.fori_loop(0, 128, fstep(0), (zero,) * _L,
                                  unroll=8)
            acc_i = lax.fori_loop(0, 128, fstep(128), (zero,) * _L,
                                  unroll=8)
            for d in range(_L):
                plsc.store_scatter(
                    obuf, [row_ids, jnp.full((_L,), d, jnp.int32)], acc_r[d])
                plsc.store_scatter(
                    obuf, [row_ids, jnp.full((_L,), d + _L, jnp.int32)],
                    acc_i[d])
        pltpu.sync_copy(obuf, out_hbm.at[pl.ds(base + c * _CH, _CH)])
        return ()

    lax.fori_loop(0, rows_per_w // _CH, do_chunk, ())


def kernel(inputs, lookup_table_real, lookup_table_imag):
    B, T, F2 = inputs.shape
    half = lookup_table_real.shape[1]
    D = 2 * half
    rows = B * T
    x = inputs.reshape(rows, F2)

    w = jnp.concatenate([lookup_table_real, lookup_table_imag], axis=0)
    wb = jnp.broadcast_to(w.reshape(F2 * half, 1), (F2 * half, _L))
    wb = jnp.asarray(wb)

    mesh = plsc.VectorSubcoreMesh(core_axis_name="c", subcore_axis_name="s")

    fe = pl.kernel(
        _sc_body,
        out_type=jax.ShapeDtypeStruct((rows, D), jnp.float32),
        mesh=mesh,
        scratch_types=[
            pltpu.VMEM((_CH, F2), jnp.float32),
            pltpu.VMEM((_CH, D), jnp.float32),
            pltpu.VMEM((F2 * half, _L), jnp.float32),
        ],
        compiler_params=pltpu.CompilerParams(
            use_tc_tiling_on_sc=False, needs_layout_passes=False),
    )
    out = fe(x, wb)
    return out.reshape(B, T, D)


# speedup vs baseline: 18.3556x; 18.3556x over previous
import jax
import jax.numpy as jnp
from jax.experimental import pallas as pl


def _fe_block(x_ref, w_ref, o_ref):
    o_ref[...] = jnp.dot(x_ref[...], w_ref[...],
                         preferred_element_type=jnp.float32)


def kernel(inputs, lookup_table_real, lookup_table_imag):
    B, T, F2 = inputs.shape
    half = lookup_table_real.shape[1]
    D = 2 * half
    F = F2 // 2
    rows = B * T
    x = inputs.reshape(rows, F2)

    w = jnp.zeros((F2, D), jnp.float32)
    w = w.at[:F, :half].set(lookup_table_real)
    w = w.at[F:, half:].set(lookup_table_imag)

    R = 10240
    assert rows % R == 0
    grid = (rows // R,)

    out = pl.pallas_call(
        _fe_block,
        grid=grid,
        in_specs=[
            pl.BlockSpec((R, F2), lambda i: (i, 0)),
            pl.BlockSpec((F2, D), lambda i: (0, 0)),
        ],
        out_specs=pl.BlockSpec((R, D), lambda i: (i, 0)),
        out_shape=jax.ShapeDtypeStruct((rows, D), jnp.float32),
    )(x, w)
    return out.reshape(B, T, D)


# speedup vs baseline: 18.6379x; 1.0154x over previous
import jax
import jax.numpy as jnp
from jax.experimental import pallas as pl


def _fe_block(x_ref, w_ref, o_ref):
    o_ref[...] = jnp.dot(x_ref[...], w_ref[...],
                         preferred_element_type=jnp.float32)


def kernel(inputs, lookup_table_real, lookup_table_imag):
    B, T, F2 = inputs.shape
    half = lookup_table_real.shape[1]
    D = 2 * half
    F = F2 // 2
    rows = B * T
    x = inputs.reshape(rows, F2)

    z = jnp.zeros((F, half), jnp.float32)
    w = jnp.concatenate(
        [jnp.concatenate([lookup_table_real, z], axis=1),
         jnp.concatenate([z, lookup_table_imag], axis=1)], axis=0)

    R = 10240
    assert rows % R == 0
    grid = (rows // R,)

    out = pl.pallas_call(
        _fe_block,
        grid=grid,
        in_specs=[
            pl.BlockSpec((R, F2), lambda i: (i, 0)),
            pl.BlockSpec((F2, D), lambda i: (0, 0)),
        ],
        out_specs=pl.BlockSpec((R, D), lambda i: (i, 0)),
        out_shape=jax.ShapeDtypeStruct((rows, D), jnp.float32),
    )(x, w)
    return out.reshape(B, T, D)
